# Initial kernel scaffold; baseline (speedup 1.0000x reference)
import jax
import jax.numpy as jnp
from jax.experimental import pallas as pl


def kernel(x, emb):
    raise NotImplementedError("write your pallas kernel here")


# speedup vs baseline: 1.4435x; 1.4435x over previous
import jax
import jax.numpy as jnp
from jax.experimental import pallas as pl

_BLK_N = 512


def _add_kernel(x_ref, emb_ref, o_ref):
    o_ref[...] = x_ref[...] + emb_ref[...]


def kernel(x, emb):
    B, N, D = x.shape
    nb = N // _BLK_N
    return pl.pallas_call(
        _add_kernel,
        grid=(nb, B),
        in_specs=[
            pl.BlockSpec((1, _BLK_N, D), lambda i, b: (b, i, 0)),
            pl.BlockSpec((_BLK_N, D), lambda i, b: (i, 0)),
        ],
        out_specs=pl.BlockSpec((1, _BLK_N, D), lambda i, b: (b, i, 0)),
        out_shape=jax.ShapeDtypeStruct((B, N, D), x.dtype),
    )(x, emb[:N])


# speedup vs baseline: 1.6849x; 1.1673x over previous
import jax
import jax.numpy as jnp
from jax.experimental import pallas as pl

_BLK_N = 1024


def _add_kernel(x_ref, emb_ref, o_ref):
    o_ref[...] = x_ref[...] + emb_ref[...]


def kernel(x, emb):
    B, N, D = x.shape
    nb = N // _BLK_N
    return pl.pallas_call(
        _add_kernel,
        grid=(nb, B),
        in_specs=[
            pl.BlockSpec((1, _BLK_N, D), lambda i, b: (b, i, 0)),
            pl.BlockSpec((_BLK_N, D), lambda i, b: (i, 0)),
        ],
        out_specs=pl.BlockSpec((1, _BLK_N, D), lambda i, b: (b, i, 0)),
        out_shape=jax.ShapeDtypeStruct((B, N, D), x.dtype),
    )(x, emb[:N])


# speedup vs baseline: 1.7954x; 1.0655x over previous
import jax
import jax.numpy as jnp
from jax.experimental import pallas as pl

_BLK_N = 2048


def _add_kernel(x_ref, emb_ref, o_ref):
    o_ref[...] = x_ref[...] + emb_ref[...]


def kernel(x, emb):
    B, N, D = x.shape
    nb = N // _BLK_N
    return pl.pallas_call(
        _add_kernel,
        grid=(nb, B),
        in_specs=[
            pl.BlockSpec((1, _BLK_N, D), lambda i, b: (b, i, 0)),
            pl.BlockSpec((_BLK_N, D), lambda i, b: (i, 0)),
        ],
        out_specs=pl.BlockSpec((1, _BLK_N, D), lambda i, b: (b, i, 0)),
        out_shape=jax.ShapeDtypeStruct((B, N, D), x.dtype),
    )(x, emb[:N])
